# baseline (speedup 1.0000x reference)
import functools

import numpy as np
import jax
import jax.numpy as jnp
from jax import lax
from jax.experimental import pallas as pl
from jax.experimental.pallas import tpu as pltpu
from jax.experimental.pallas import tpu_sc as plsc

_NCLASSES = 80
_MAX_OUT = 100
_IOU_THR = 0.5
_NEG = np.float32(-1e10)
_N = 6300
_NPAD = 6400
_ROWS, _COLS = 8, 800
_L = 16
_BLK = 16
_NBLK = _NPAD // (_L * _BLK)
_OUTW = (_MAX_OUT + 1) * _L

_ANCHORS = np.array([
    [[0.024, 0.031], [0.038, 0.072], [0.079, 0.055]],
    [[0.072, 0.147], [0.149, 0.108], [0.142, 0.286]],
    [[0.279, 0.216], [0.375, 0.476], [0.897, 0.784]],
], dtype=np.float32)


def _build_consts():
    gx = np.zeros(_NPAD, np.float32)
    gy = np.zeros(_NPAD, np.float32)
    aw = np.ones(_NPAD, np.float32)
    ah = np.ones(_NPAD, np.float32)
    gsv = np.ones(_NPAD, np.float32)
    vmask = np.zeros(_NPAD, np.float32)
    base = 0
    for gs, anc in ((10, _ANCHORS[2]), (20, _ANCHORS[1]), (40, _ANCHORS[0])):
        n = gs * gs * 3
        ii, jj, aa = np.meshgrid(np.arange(gs), np.arange(gs), np.arange(3),
                                 indexing="ij")
        gx[base:base + n] = jj.ravel().astype(np.float32)
        gy[base:base + n] = ii.ravel().astype(np.float32)
        aw[base:base + n] = anc[aa.ravel(), 0]
        ah[base:base + n] = anc[aa.ravel(), 1]
        gsv[base:base + n] = float(gs)
        base += n
    vmask[:_N] = 1.0
    return np.stack([gx, gy, aw, ah, gsv, vmask]).reshape(6, _ROWS, _COLS)

_CONSTS = _build_consts()


def _decode_kernel(feat_ref, const_ref, dec_ref):
    gx = const_ref[0]
    gy = const_ref[1]
    aw = const_ref[2]
    ah = const_ref[3]
    gsv = const_ref[4]
    vmaskb = const_ref[5] > 0.0

    x = (feat_ref[0] + gx) / gsv
    y = (feat_ref[1] + gy) / gsv
    w = jnp.exp(feat_ref[2]) * aw
    h = jnp.exp(feat_ref[3]) * ah
    conf = feat_ref[4]
    x1 = x - w * 0.5
    x2 = x + w * 0.5
    y1 = y - h * 0.5
    y2 = y + h * 0.5

    best = feat_ref[5]
    bidx = jnp.zeros((_ROWS, _COLS), jnp.float32)
    for k in range(1, _NCLASSES):
        c = feat_ref[5 + k]
        upd = c > best
        best = jnp.where(upd, c, best)
        bidx = jnp.where(upd, jnp.float32(k), bidx)

    cmax_hi = jnp.max(jnp.where(vmaskb, best, jnp.float32(-3e38)))
    cmax_lo = jnp.min(jnp.where(vmaskb, best, jnp.float32(3e38)))
    s0 = conf * jnp.where(conf > 0.0, cmax_hi, cmax_lo)
    s0 = jnp.where((s0 > 0.0) & vmaskb, s0, _NEG)

    dec_ref[0] = x1
    dec_ref[1] = y1
    dec_ref[2] = x2
    dec_ref[3] = y2
    dec_ref[4] = bidx
    dec_ref[5] = s0


def _sc_nms(dec_hbm, out_hbm, x1v, y1v, x2v, y2v, clsv, scorev,
            outacc_v, tmp_v):
    cid = lax.axis_index("c")
    tid = lax.axis_index("s")

    @pl.when((cid == 0) & (tid == 0))
    def _tile0():
        for r, ref in enumerate((x1v, y1v, x2v, y2v, clsv, scorev)):
            pltpu.sync_copy(dec_hbm.at[pl.ds(r * _NPAD, _NPAD)], ref)

        iota = lax.iota(jnp.int32, _L)
        negv = jnp.float32(_NEG)

        def perm(v, ix):
            if v.dtype == jnp.int32:
                tmp_v[...] = plsc.bitcast(v, jnp.float32)
                return plsc.bitcast(plsc.load_gather(tmp_v, [ix]), jnp.int32)
            tmp_v[...] = v
            return plsc.load_gather(tmp_v, [ix])

        def bfly(v, op):
            for k in (1, 2, 4, 8):
                v = op(v, perm(v, lax.bitwise_xor(iota, k)))
            return v

        def body(i, carry):
            wx1, wy1, wx2, wy2, wa, wg, nv = carry

            def scan_blk(blk, acc):
                bs, bg = acc
                b0 = blk * (_L * _BLK)
                for j in range(_BLK):
                    sl = pl.ds(b0 + j * _L, _L)
                    s = scorev[sl]
                    bx1 = x1v[sl]
                    by1 = y1v[sl]
                    bx2 = x2v[sl]
                    by2 = y2v[sl]
                    ar = (jnp.maximum(bx2 - bx1, 0.0)
                          * jnp.maximum(by2 - by1, 0.0))
                    gidx = iota + (b0 + j * _L)
                    ix1 = jnp.maximum(wx1, bx1)
                    iy1 = jnp.maximum(wy1, by1)
                    ix2 = jnp.minimum(wx2, bx2)
                    iy2 = jnp.minimum(wy2, by2)
                    inter = (jnp.maximum(ix2 - ix1, 0.0)
                             * jnp.maximum(iy2 - iy1, 0.0))
                    union = wa + ar - inter
                    upos = union > 0.0
                    iou = jnp.where(upos,
                                    inter / jnp.where(upos, union, 1.0), 0.0)
                    kill = (iou > _IOU_THR) | (gidx == wg)
                    s = jnp.where(kill, negv, s)
                    scorev[sl] = s
                    upd = s > bs
                    bs = jnp.where(upd, s, bs)
                    bg = jnp.where(upd, gidx, bg)
                return bs, bg

            bs0 = jnp.full((_L,), -3e38, jnp.float32)
            bg0 = jnp.zeros((_L,), jnp.int32)
            bs, bg = lax.fori_loop(0, _NBLK, scan_blk, (bs0, bg0))

            mg = bfly(bs, jnp.maximum)
            gw = bfly(jnp.where(bs == mg, bg, jnp.int32(2**31 - 1)),
                      jnp.minimum)
            nwx1 = plsc.load_gather(x1v, [gw])
            nwy1 = plsc.load_gather(y1v, [gw])
            nwx2 = plsc.load_gather(x2v, [gw])
            nwy2 = plsc.load_gather(y2v, [gw])
            nwcls = plsc.load_gather(clsv, [gw])
            nwa = (jnp.maximum(nwx2 - nwx1, 0.0)
                   * jnp.maximum(nwy2 - nwy1, 0.0))
            validv = mg > jnp.float32(_NEG * 0.5)

            zv = jnp.zeros((_L,), jnp.float32)
            orow = jnp.where(iota == 0, jnp.where(validv, nwx1, zv),
                   jnp.where(iota == 1, jnp.where(validv, nwy1, zv),
                   jnp.where(iota == 2, jnp.where(validv, nwx2, zv),
                   jnp.where(iota == 3, jnp.where(validv, nwy2, zv),
                   jnp.where(iota == 4, jnp.where(validv, mg, zv),
                   jnp.where(iota == 5, jnp.where(validv, nwcls, zv),
                             zv))))))
            outacc_v[pl.ds(i * _L, _L)] = orow

            nv = nv + jnp.where(validv, jnp.float32(1.0), jnp.float32(0.0))
            return (nwx1, nwy1, nwx2, nwy2, nwa, gw, nv)

        zv16 = jnp.zeros((_L,), jnp.float32)
        carry = (zv16, zv16, zv16, zv16, zv16,
                 jnp.full((_L,), -1, jnp.int32), zv16)
        carry = lax.fori_loop(0, _MAX_OUT, body, carry)

        nvrow = jnp.where(iota == 0, carry[6], jnp.zeros((_L,), jnp.float32))
        outacc_v[pl.ds(_MAX_OUT * _L, _L)] = nvrow
        pltpu.sync_copy(outacc_v, out_hbm)


@jax.jit
def kernel(grid0, grid1, grid2):
    parts = [grid0.reshape(-1, 85), grid1.reshape(-1, 85),
             grid2.reshape(-1, 85)]
    allf = jnp.concatenate(parts, axis=0)
    allf = jnp.pad(allf, ((0, _NPAD - _N), (0, 0)))
    feat = allf.T.reshape(85, _ROWS, _COLS)
    consts = jnp.asarray(_CONSTS)

    dec = pl.pallas_call(
        _decode_kernel,
        out_shape=jax.ShapeDtypeStruct((6, _ROWS, _COLS), jnp.float32),
        in_specs=[pl.BlockSpec(memory_space=pltpu.VMEM),
                  pl.BlockSpec(memory_space=pltpu.VMEM)],
        out_specs=pl.BlockSpec(memory_space=pltpu.VMEM),
    )(feat, consts)
    dec_flat = dec.reshape(-1)

    mesh = plsc.VectorSubcoreMesh(core_axis_name="c", subcore_axis_name="s",
                                  num_cores=2, num_subcores=16)
    nms = functools.partial(
        pl.kernel,
        out_type=jax.ShapeDtypeStruct((_OUTW,), jnp.float32),
        mesh=mesh,
        compiler_params=pltpu.CompilerParams(needs_layout_passes=False),
        scratch_types=[
            pltpu.VMEM((_NPAD,), jnp.float32),
            pltpu.VMEM((_NPAD,), jnp.float32),
            pltpu.VMEM((_NPAD,), jnp.float32),
            pltpu.VMEM((_NPAD,), jnp.float32),
            pltpu.VMEM((_NPAD,), jnp.float32),
            pltpu.VMEM((_NPAD,), jnp.float32),
            pltpu.VMEM((_OUTW,), jnp.float32),
            pltpu.VMEM((_L,), jnp.float32),
        ],
    )(_sc_nms)
    outf = nms(dec_flat)

    res = outf.reshape(_MAX_OUT + 1, _L)
    boxes = res[:_MAX_OUT, 0:4]
    scores = res[:_MAX_OUT, 4]
    cls = res[:_MAX_OUT, 5].astype(jnp.int32)
    nv = res[_MAX_OUT, 0].astype(jnp.int32)
    return (boxes[None], scores[None], cls[None], nv.reshape(1))


# speedup vs baseline: 1.9043x; 1.9043x over previous
import functools

import numpy as np
import jax
import jax.numpy as jnp
from jax import lax
from jax.experimental import pallas as pl
from jax.experimental.pallas import tpu as pltpu
from jax.experimental.pallas import tpu_sc as plsc

_NCLASSES = 80
_MAX_OUT = 100
_IOU_THR = 0.5
_NEG = np.float32(-1e10)
_N = 6300
_NPAD = 6400
_ROWS, _COLS = 8, 800
_L = 16
_CAP = _NPAD + _L
_OUTW = (_MAX_OUT + 1) * _L

_ANCHORS = np.array([
    [[0.024, 0.031], [0.038, 0.072], [0.079, 0.055]],
    [[0.072, 0.147], [0.149, 0.108], [0.142, 0.286]],
    [[0.279, 0.216], [0.375, 0.476], [0.897, 0.784]],
], dtype=np.float32)


def _build_consts():
    gx = np.zeros(_NPAD, np.float32)
    gy = np.zeros(_NPAD, np.float32)
    aw = np.ones(_NPAD, np.float32)
    ah = np.ones(_NPAD, np.float32)
    gsv = np.ones(_NPAD, np.float32)
    vmask = np.zeros(_NPAD, np.float32)
    base = 0
    for gs, anc in ((10, _ANCHORS[2]), (20, _ANCHORS[1]), (40, _ANCHORS[0])):
        n = gs * gs * 3
        ii, jj, aa = np.meshgrid(np.arange(gs), np.arange(gs), np.arange(3),
                                 indexing="ij")
        gx[base:base + n] = jj.ravel().astype(np.float32)
        gy[base:base + n] = ii.ravel().astype(np.float32)
        aw[base:base + n] = anc[aa.ravel(), 0]
        ah[base:base + n] = anc[aa.ravel(), 1]
        gsv[base:base + n] = float(gs)
        base += n
    vmask[:_N] = 1.0
    return np.stack([gx, gy, aw, ah, gsv, vmask]).reshape(6, _ROWS, _COLS)

_CONSTS = _build_consts()


def _decode_kernel(feat_ref, const_ref, dec_ref):
    gx = const_ref[0]
    gy = const_ref[1]
    aw = const_ref[2]
    ah = const_ref[3]
    gsv = const_ref[4]
    vmaskb = const_ref[5] > 0.0

    x = (feat_ref[0] + gx) / gsv
    y = (feat_ref[1] + gy) / gsv
    w = jnp.exp(feat_ref[2]) * aw
    h = jnp.exp(feat_ref[3]) * ah
    conf = feat_ref[4]
    x1 = x - w * 0.5
    x2 = x + w * 0.5
    y1 = y - h * 0.5
    y2 = y + h * 0.5

    best = feat_ref[5]
    bidx = jnp.zeros((_ROWS, _COLS), jnp.float32)
    for k in range(1, _NCLASSES):
        c = feat_ref[5 + k]
        upd = c > best
        best = jnp.where(upd, c, best)
        bidx = jnp.where(upd, jnp.float32(k), bidx)

    cmax_hi = jnp.max(jnp.where(vmaskb, best, jnp.float32(-3e38)))
    cmax_lo = jnp.min(jnp.where(vmaskb, best, jnp.float32(3e38)))
    s0 = conf * jnp.where(conf > 0.0, cmax_hi, cmax_lo)
    s0 = jnp.where((s0 > 0.0) & vmaskb, s0, _NEG)

    dec_ref[0] = x1
    dec_ref[1] = y1
    dec_ref[2] = x2
    dec_ref[3] = y2
    dec_ref[4] = bidx
    dec_ref[5] = s0


def _sc_nms(dec_hbm, out_hbm, x1v, y1v, x2v, y2v, clsv, scorev, gidxv,
            outacc_v, tmp_v):
    cid = lax.axis_index("c")
    tid = lax.axis_index("s")

    @pl.when((cid == 0) & (tid == 0))
    def _tile0():
        for r, ref in enumerate((x1v, y1v, x2v, y2v, clsv, scorev)):
            pltpu.sync_copy(dec_hbm.at[pl.ds(r * _NPAD, _NPAD)],
                            ref.at[pl.ds(0, _NPAD)])

        iota = lax.iota(jnp.int32, _L)
        negv = jnp.float32(_NEG)
        livethr = jnp.float32(_NEG * 0.5)

        def perm(v, ix):
            if v.dtype == jnp.int32:
                tmp_v[...] = plsc.bitcast(v, jnp.float32)
                return plsc.bitcast(plsc.load_gather(tmp_v, [ix]), jnp.int32)
            tmp_v[...] = v
            return plsc.load_gather(tmp_v, [ix])

        def bfly(v, op):
            for k in (1, 2, 4, 8):
                v = op(v, perm(v, lax.bitwise_xor(iota, k)))
            return v

        def comp(c, off):
            sl = pl.ds(c * _L, _L)
            s = scorev[sl]
            m = s > livethr
            ps = plsc.cumsum(jnp.where(m, 1, 0))
            pos = (off - 1) + ps
            plsc.store_scatter(scorev, [pos], s, mask=m)
            plsc.store_scatter(x1v, [pos], x1v[sl], mask=m)
            plsc.store_scatter(y1v, [pos], y1v[sl], mask=m)
            plsc.store_scatter(x2v, [pos], x2v[sl], mask=m)
            plsc.store_scatter(y2v, [pos], y2v[sl], mask=m)
            plsc.store_scatter(clsv, [pos], clsv[sl], mask=m)
            plsc.store_scatter(gidxv, [pos], iota + c * _L, mask=m)
            return off + ps[_L - 1]

        live = lax.fori_loop(0, _NPAD // _L, comp, jnp.int32(0))

        al = (live // _L) * _L
        tl = pl.ds(al, _L)
        lm = iota >= (live - al)
        scorev[tl] = jnp.where(lm, negv, scorev[tl])
        gidxv[tl] = jnp.where(lm, jnp.int32(0), gidxv[tl])
        nch = (live + (_L - 1)) // _L

        def body(i, carry):
            wx1, wy1, wx2, wy2, wa, wg, nv = carry

            def scan_chunk(c, acc):
                bs, bg, bp = acc
                sl = pl.ds(c * _L, _L)
                s = scorev[sl]
                bx1 = x1v[sl]
                by1 = y1v[sl]
                bx2 = x2v[sl]
                by2 = y2v[sl]
                ar = (jnp.maximum(bx2 - bx1, 0.0)
                      * jnp.maximum(by2 - by1, 0.0))
                gidx = gidxv[sl]
                cpos = iota + c * _L
                ix1 = jnp.maximum(wx1, bx1)
                iy1 = jnp.maximum(wy1, by1)
                ix2 = jnp.minimum(wx2, bx2)
                iy2 = jnp.minimum(wy2, by2)
                inter = (jnp.maximum(ix2 - ix1, 0.0)
                         * jnp.maximum(iy2 - iy1, 0.0))
                union = wa + ar - inter
                upos = union > 0.0
                iou = jnp.where(upos,
                                inter / jnp.where(upos, union, 1.0), 0.0)
                kill = (iou > _IOU_THR) | (gidx == wg)
                s = jnp.where(kill, negv, s)
                scorev[sl] = s
                upd = s > bs
                bs = jnp.where(upd, s, bs)
                bg = jnp.where(upd, gidx, bg)
                bp = jnp.where(upd, cpos, bp)
                return bs, bg, bp

            bs0 = jnp.full((_L,), -3e38, jnp.float32)
            bg0 = jnp.zeros((_L,), jnp.int32)
            bs, bg, bp = lax.fori_loop(0, nch, scan_chunk, (bs0, bg0, bg0))

            mg = bfly(bs, jnp.maximum)
            big = jnp.int32(2**31 - 1)
            gw = jnp.where(bs == mg, bg, big)
            pw = jnp.where(bs == mg, bp, jnp.int32(0))
            for k in (1, 2, 4, 8):
                g2 = perm(gw, lax.bitwise_xor(iota, k))
                p2 = perm(pw, lax.bitwise_xor(iota, k))
                take = g2 < gw
                gw = jnp.where(take, g2, gw)
                pw = jnp.where(take, p2, pw)
            nwx1 = plsc.load_gather(x1v, [pw])
            nwy1 = plsc.load_gather(y1v, [pw])
            nwx2 = plsc.load_gather(x2v, [pw])
            nwy2 = plsc.load_gather(y2v, [pw])
            nwcls = plsc.load_gather(clsv, [pw])
            nwa = (jnp.maximum(nwx2 - nwx1, 0.0)
                   * jnp.maximum(nwy2 - nwy1, 0.0))
            validv = mg > jnp.float32(_NEG * 0.5)

            zv = jnp.zeros((_L,), jnp.float32)
            orow = jnp.where(iota == 0, jnp.where(validv, nwx1, zv),
                   jnp.where(iota == 1, jnp.where(validv, nwy1, zv),
                   jnp.where(iota == 2, jnp.where(validv, nwx2, zv),
                   jnp.where(iota == 3, jnp.where(validv, nwy2, zv),
                   jnp.where(iota == 4, jnp.where(validv, mg, zv),
                   jnp.where(iota == 5, jnp.where(validv, nwcls, zv),
                             zv))))))
            outacc_v[pl.ds(i * _L, _L)] = orow

            nv = nv + jnp.where(validv, jnp.float32(1.0), jnp.float32(0.0))
            return (nwx1, nwy1, nwx2, nwy2, nwa, gw, nv)

        zv16 = jnp.zeros((_L,), jnp.float32)
        carry = (zv16, zv16, zv16, zv16, zv16,
                 jnp.full((_L,), -1, jnp.int32), zv16)
        carry = lax.fori_loop(0, _MAX_OUT, body, carry)

        nvrow = jnp.where(iota == 0, carry[6], jnp.zeros((_L,), jnp.float32))
        outacc_v[pl.ds(_MAX_OUT * _L, _L)] = nvrow
        pltpu.sync_copy(outacc_v, out_hbm)


@jax.jit
def kernel(grid0, grid1, grid2):
    parts = [grid0.reshape(-1, 85), grid1.reshape(-1, 85),
             grid2.reshape(-1, 85)]
    allf = jnp.concatenate(parts, axis=0)
    allf = jnp.pad(allf, ((0, _NPAD - _N), (0, 0)))
    feat = allf.T.reshape(85, _ROWS, _COLS)
    consts = jnp.asarray(_CONSTS)

    dec = pl.pallas_call(
        _decode_kernel,
        out_shape=jax.ShapeDtypeStruct((6, _ROWS, _COLS), jnp.float32),
        in_specs=[pl.BlockSpec(memory_space=pltpu.VMEM),
                  pl.BlockSpec(memory_space=pltpu.VMEM)],
        out_specs=pl.BlockSpec(memory_space=pltpu.VMEM),
    )(feat, consts)
    dec_flat = dec.reshape(-1)

    mesh = plsc.VectorSubcoreMesh(core_axis_name="c", subcore_axis_name="s",
                                  num_cores=2, num_subcores=16)
    nms = functools.partial(
        pl.kernel,
        out_type=jax.ShapeDtypeStruct((_OUTW,), jnp.float32),
        mesh=mesh,
        compiler_params=pltpu.CompilerParams(needs_layout_passes=False),
        scratch_types=[
            pltpu.VMEM((_CAP,), jnp.float32),
            pltpu.VMEM((_CAP,), jnp.float32),
            pltpu.VMEM((_CAP,), jnp.float32),
            pltpu.VMEM((_CAP,), jnp.float32),
            pltpu.VMEM((_CAP,), jnp.float32),
            pltpu.VMEM((_CAP,), jnp.float32),
            pltpu.VMEM((_CAP,), jnp.int32),
            pltpu.VMEM((_OUTW,), jnp.float32),
            pltpu.VMEM((_L,), jnp.float32),
        ],
    )(_sc_nms)
    outf = nms(dec_flat)

    res = outf.reshape(_MAX_OUT + 1, _L)
    boxes = res[:_MAX_OUT, 0:4]
    scores = res[:_MAX_OUT, 4]
    cls = res[:_MAX_OUT, 5].astype(jnp.int32)
    nv = res[_MAX_OUT, 0].astype(jnp.int32)
    return (boxes[None], scores[None], cls[None], nv.reshape(1))
